# baseline (speedup 1.0000x reference)
import functools

import jax
import jax.numpy as jnp
from jax import lax
from jax.experimental import pallas as pl
from jax.experimental.pallas import tpu as pltpu
from jax.experimental.pallas import tpu_sc as plsc

_N = 10000
_D = 128
_E = 320000
_NC = 2
_NS = 16
_NW = _NC * _NS
_EPW = _E // _NW
_K = 80
_NCH = _EPW // _K
_CPP0 = 64
_CPP1 = _NCH - _CPP0
_RPT = 624
_RTAIL = _N - _NS * _RPT
_BLK = 1000
_GRID = _N // _BLK

def _segsum_body(h_hbm, src_hbm, dst_hbm, zeros_hbm, out_hbm, src_v, dst_v, rows_v, m_sh, gsem, ssem):
    c = lax.axis_index("c")
    s = lax.axis_index("s")
    wid = c * _NS + s
    pltpu.sync_copy(zeros_hbm.at[pl.ds(0, _RPT)], m_sh.at[pl.ds(s * _RPT, _RPT)])

    @pl.when(s == _NS - 1)
    def _():
        pltpu.sync_copy(
            zeros_hbm.at[pl.ds(_RPT, _RTAIL)],
            m_sh.at[pl.ds(_NS * _RPT, _RTAIL)],
        )
    pltpu.sync_copy(src_hbm.at[wid, pl.ds(0, _CPP0)], src_v.at[pl.ds(0, _CPP0)])
    pltpu.sync_copy(dst_hbm.at[wid, pl.ds(0, _CPP0)], dst_v.at[pl.ds(0, _CPP0)])
    plsc.subcore_barrier()

    def _buf(j):
        b = lax.rem(j, jnp.int32(3))
        return b

    def _issue_gather(j):
        b = _buf(j)
        pltpu.async_copy(h_hbm.at[src_v.at[j]], rows_v.at[b], gsem.at[b])

    def _wait_gather(j):
        b = _buf(j)
        pltpu.make_async_copy(h_hbm.at[src_v.at[j]], rows_v.at[b], gsem.at[b]).wait()

    def _issue_scatter(j):
        b = _buf(j)
        pltpu.async_copy(rows_v.at[b], m_sh.at[dst_v.at[j]], ssem.at[b], add=True)

    def _wait_scatter(j):
        b = _buf(j)
        pltpu.make_async_copy(rows_v.at[b], m_sh.at[dst_v.at[j]], ssem.at[b]).wait()

    def body(j, carry):
        _wait_gather(j)

        @pl.when(j >= 2)
        def _():
            _wait_scatter(j - 2)

        @pl.when(j + 1 < carry)
        def _():
            _issue_gather(j + 1)

        _issue_scatter(j)
        return carry

    for p, cpp in enumerate((_CPP0, _CPP1)):
        if p > 0:
            pltpu.sync_copy(
                src_hbm.at[wid, pl.ds(_CPP0, _CPP1)], src_v.at[pl.ds(0, _CPP1)]
            )
            pltpu.sync_copy(
                dst_hbm.at[wid, pl.ds(_CPP0, _CPP1)], dst_v.at[pl.ds(0, _CPP1)]
            )
        _issue_gather(jnp.int32(0))
        lax.fori_loop(0, cpp, body, jnp.int32(cpp))
        _wait_scatter(jnp.int32(cpp - 2))
        _wait_scatter(jnp.int32(cpp - 1))
    plsc.subcore_barrier()
    pltpu.sync_copy(m_sh.at[pl.ds(s * _RPT, _RPT)], out_hbm.at[c, pl.ds(s * _RPT, _RPT)])

    @pl.when(s == _NS - 1)
    def _():
        pltpu.sync_copy(
            m_sh.at[pl.ds(_NS * _RPT, _RTAIL)],
            out_hbm.at[c, pl.ds(_NS * _RPT, _RTAIL)],
        )


@functools.cache
def _make_segsum():
    mesh = plsc.VectorSubcoreMesh(
        core_axis_name="c", subcore_axis_name="s", num_cores=_NC, num_subcores=_NS
    )
    return pl.kernel(
        _segsum_body,
        out_type=jax.ShapeDtypeStruct((_NC, _N, _D), jnp.float32),
        mesh=mesh,
        scratch_types=[
            pltpu.VMEM((_CPP0, _K), jnp.int32),
            pltpu.VMEM((_CPP0, _K), jnp.int32),
            pltpu.VMEM((3, _K, _D), jnp.float32),
            pltpu.VMEM_SHARED((_N, _D), jnp.float32),
            pltpu.SemaphoreType.DMA((3,)),
            pltpu.SemaphoreType.DMA((3,)),
        ],
    )


def _proj_body(x_ref, w_ref, b_ref, o_ref):
    o_ref[...] = (
        jnp.dot(x_ref[...], w_ref[...], preferred_element_type=jnp.float32) + b_ref[...]
    )


_proj = pl.pallas_call(
    _proj_body,
    grid=(_GRID,),
    in_specs=[
        pl.BlockSpec((_BLK, _D), lambda i: (i, 0)),
        pl.BlockSpec((_D, _D), lambda i: (0, 0)),
        pl.BlockSpec((1, _D), lambda i: (0, 0)),
    ],
    out_specs=pl.BlockSpec((_BLK, _D), lambda i: (i, 0)),
    out_shape=jax.ShapeDtypeStruct((_N, _D), jnp.float32),
)


def _gru_body(mp_ref, h_ref, wih_ref, bih_ref, whh_ref, bhh_ref, wn_ref, bn_ref, o_ref):
    m = mp_ref[0] + mp_ref[1]
    h = h_ref[...]
    gi = jnp.dot(m, wih_ref[...], preferred_element_type=jnp.float32) + bih_ref[...]
    gh = jnp.dot(h, whh_ref[...], preferred_element_type=jnp.float32) + bhh_ref[...]
    r = jax.nn.sigmoid(gi[:, :_D] + gh[:, :_D])
    z = jax.nn.sigmoid(gi[:, _D:2 * _D] + gh[:, _D:2 * _D])
    n = jnp.tanh(gi[:, 2 * _D:] + r * gh[:, 2 * _D:])
    x = (1.0 - z) * n + z * h
    e = jnp.where(x > 0, x, jnp.exp(x) - 1.0)
    o_ref[...] = (
        jnp.dot(e, wn_ref[...], preferred_element_type=jnp.float32) + bn_ref[...]
    )


_gru = pl.pallas_call(
    _gru_body,
    grid=(_GRID,),
    in_specs=[
        pl.BlockSpec((_NC, _BLK, _D), lambda i: (0, i, 0)),
        pl.BlockSpec((_BLK, _D), lambda i: (i, 0)),
        pl.BlockSpec((_D, 3 * _D), lambda i: (0, 0)),
        pl.BlockSpec((1, 3 * _D), lambda i: (0, 0)),
        pl.BlockSpec((_D, 3 * _D), lambda i: (0, 0)),
        pl.BlockSpec((1, 3 * _D), lambda i: (0, 0)),
        pl.BlockSpec((_D, _D), lambda i: (0, 0)),
        pl.BlockSpec((1, _D), lambda i: (0, 0)),
    ],
    out_specs=pl.BlockSpec((_BLK, _D), lambda i: (i, 0)),
    out_shape=jax.ShapeDtypeStruct((_N, _D), jnp.float32),
)


def kernel(x_0, edge_index_0, x_1, edge_index_1, Wlin, blin, Wih, bih, Whh, bhh, fcW, fcb):
    zeros = jnp.zeros((_RPT + _RTAIL, _D), jnp.float32)
    _segsum = _make_segsum()
    outs = []
    for g, (x, ei) in enumerate(((x_0, edge_index_0), (x_1, edge_index_1))):
        src = ei[0].reshape(_NW, _NCH, _K)
        dst = ei[1].reshape(_NW, _NCH, _K)
        i0, i1 = 2 * g, 2 * g + 1
        h = _proj(x, Wlin[i0], blin[i0].reshape(1, _D))
        mp = _segsum(h, src, dst, zeros)
        h = _gru(
            mp, h,
            Wih[i0], bih[i0].reshape(1, 3 * _D),
            Whh[i0], bhh[i0].reshape(1, 3 * _D),
            Wlin[i1], blin[i1].reshape(1, _D),
        )
        mp = _segsum(h, src, dst, zeros)
        outs.append(
            _gru(
                mp, h,
                Wih[i1], bih[i1].reshape(1, 3 * _D),
                Whh[i1], bhh[i1].reshape(1, 3 * _D),
                fcW, fcb.reshape(1, _D),
            )
        )
    return jnp.concatenate(outs, axis=0)


# speedup vs baseline: 1.4438x; 1.4438x over previous
import functools

import jax
import jax.numpy as jnp
from jax import lax
from jax.experimental import pallas as pl
from jax.experimental.pallas import tpu as pltpu
from jax.experimental.pallas import tpu_sc as plsc

_N = 10000
_D = 128
_E = 320000
_NC = 2
_NS = 16
_NW = _NC * _NS
_EPW = _E // _NW
_K = 80
_NCH = _EPW // _K
_CPP0 = 64
_CPP1 = _NCH - _CPP0
_RPT = 624
_RTAIL = _N - _NS * _RPT
_BLK = 1000
_GRID = _N // _BLK

def _segsum_body(h_hbm, src_hbm, dst_hbm, zeros_hbm, out_hbm, src_v, dst_v, rows_v, m_sh, gsem, ssem):
    c = lax.axis_index("c")
    s = lax.axis_index("s")
    wid = c * _NS + s
    pltpu.sync_copy(zeros_hbm.at[pl.ds(0, _RPT)], m_sh.at[pl.ds(s * _RPT, _RPT)])

    @pl.when(s == _NS - 1)
    def _():
        pltpu.sync_copy(
            zeros_hbm.at[pl.ds(_RPT, _RTAIL)],
            m_sh.at[pl.ds(_NS * _RPT, _RTAIL)],
        )
    pltpu.sync_copy(src_hbm.at[wid, pl.ds(0, _CPP0)], src_v.at[pl.ds(0, _CPP0)])
    pltpu.sync_copy(dst_hbm.at[wid, pl.ds(0, _CPP0)], dst_v.at[pl.ds(0, _CPP0)])
    plsc.subcore_barrier()

    def _buf(j):
        b = lax.rem(j, jnp.int32(3))
        return b

    def _issue_gather(j):
        b = _buf(j)
        pltpu.async_copy(h_hbm.at[src_v.at[j]], rows_v.at[b], gsem.at[b])

    def _wait_gather(j):
        b = _buf(j)
        pltpu.make_async_copy(h_hbm.at[src_v.at[j]], rows_v.at[b], gsem.at[b]).wait()

    def _issue_scatter(j):
        b = _buf(j)
        pltpu.async_copy(rows_v.at[b], m_sh.at[dst_v.at[j]], ssem.at[b], add=True)

    def _wait_scatter(j):
        b = _buf(j)
        pltpu.make_async_copy(rows_v.at[b], m_sh.at[dst_v.at[j]], ssem.at[b]).wait()

    def body(j, carry):
        _wait_gather(j)

        @pl.when(j >= 1)
        def _():
            _wait_scatter(j - 1)

        @pl.when(j + 2 < carry)
        def _():
            _issue_gather(j + 2)

        _issue_scatter(j)
        return carry

    for p, cpp in enumerate((_CPP0, _CPP1)):
        if p > 0:
            pltpu.sync_copy(
                src_hbm.at[wid, pl.ds(_CPP0, _CPP1)], src_v.at[pl.ds(0, _CPP1)]
            )
            pltpu.sync_copy(
                dst_hbm.at[wid, pl.ds(_CPP0, _CPP1)], dst_v.at[pl.ds(0, _CPP1)]
            )
        _issue_gather(jnp.int32(0))
        _issue_gather(jnp.int32(1))
        lax.fori_loop(0, cpp, body, jnp.int32(cpp))
        _wait_scatter(jnp.int32(cpp - 1))
    plsc.subcore_barrier()
    pltpu.sync_copy(m_sh.at[pl.ds(s * _RPT, _RPT)], out_hbm.at[c, pl.ds(s * _RPT, _RPT)])

    @pl.when(s == _NS - 1)
    def _():
        pltpu.sync_copy(
            m_sh.at[pl.ds(_NS * _RPT, _RTAIL)],
            out_hbm.at[c, pl.ds(_NS * _RPT, _RTAIL)],
        )


@functools.cache
def _make_segsum():
    mesh = plsc.VectorSubcoreMesh(
        core_axis_name="c", subcore_axis_name="s", num_cores=_NC, num_subcores=_NS
    )
    return pl.kernel(
        _segsum_body,
        out_type=jax.ShapeDtypeStruct((_NC, _N, _D), jnp.float32),
        mesh=mesh,
        scratch_types=[
            pltpu.VMEM((_CPP0, _K), jnp.int32),
            pltpu.VMEM((_CPP0, _K), jnp.int32),
            pltpu.VMEM((3, _K, _D), jnp.float32),
            pltpu.VMEM_SHARED((_N, _D), jnp.float32),
            pltpu.SemaphoreType.DMA((3,)),
            pltpu.SemaphoreType.DMA((3,)),
        ],
    )


def _proj_body(x_ref, w_ref, b_ref, o_ref):
    o_ref[...] = (
        jnp.dot(x_ref[...], w_ref[...], preferred_element_type=jnp.float32) + b_ref[...]
    )


_proj = pl.pallas_call(
    _proj_body,
    grid=(_GRID,),
    in_specs=[
        pl.BlockSpec((_BLK, _D), lambda i: (i, 0)),
        pl.BlockSpec((_D, _D), lambda i: (0, 0)),
        pl.BlockSpec((1, _D), lambda i: (0, 0)),
    ],
    out_specs=pl.BlockSpec((_BLK, _D), lambda i: (i, 0)),
    out_shape=jax.ShapeDtypeStruct((_N, _D), jnp.float32),
)


def _gru_body(mp_ref, h_ref, wih_ref, bih_ref, whh_ref, bhh_ref, wn_ref, bn_ref, o_ref):
    m = mp_ref[0] + mp_ref[1]
    h = h_ref[...]
    gi = jnp.dot(m, wih_ref[...], preferred_element_type=jnp.float32) + bih_ref[...]
    gh = jnp.dot(h, whh_ref[...], preferred_element_type=jnp.float32) + bhh_ref[...]
    r = jax.nn.sigmoid(gi[:, :_D] + gh[:, :_D])
    z = jax.nn.sigmoid(gi[:, _D:2 * _D] + gh[:, _D:2 * _D])
    n = jnp.tanh(gi[:, 2 * _D:] + r * gh[:, 2 * _D:])
    x = (1.0 - z) * n + z * h
    e = jnp.where(x > 0, x, jnp.exp(x) - 1.0)
    o_ref[...] = (
        jnp.dot(e, wn_ref[...], preferred_element_type=jnp.float32) + bn_ref[...]
    )


_gru = pl.pallas_call(
    _gru_body,
    grid=(_GRID,),
    in_specs=[
        pl.BlockSpec((_NC, _BLK, _D), lambda i: (0, i, 0)),
        pl.BlockSpec((_BLK, _D), lambda i: (i, 0)),
        pl.BlockSpec((_D, 3 * _D), lambda i: (0, 0)),
        pl.BlockSpec((1, 3 * _D), lambda i: (0, 0)),
        pl.BlockSpec((_D, 3 * _D), lambda i: (0, 0)),
        pl.BlockSpec((1, 3 * _D), lambda i: (0, 0)),
        pl.BlockSpec((_D, _D), lambda i: (0, 0)),
        pl.BlockSpec((1, _D), lambda i: (0, 0)),
    ],
    out_specs=pl.BlockSpec((_BLK, _D), lambda i: (i, 0)),
    out_shape=jax.ShapeDtypeStruct((_N, _D), jnp.float32),
)


def kernel(x_0, edge_index_0, x_1, edge_index_1, Wlin, blin, Wih, bih, Whh, bhh, fcW, fcb):
    zeros = jnp.zeros((_RPT + _RTAIL, _D), jnp.float32)
    _segsum = _make_segsum()
    outs = []
    for g, (x, ei) in enumerate(((x_0, edge_index_0), (x_1, edge_index_1))):
        src = ei[0].reshape(_NW, _NCH, _K)
        dst = ei[1].reshape(_NW, _NCH, _K)
        i0, i1 = 2 * g, 2 * g + 1
        h = _proj(x, Wlin[i0], blin[i0].reshape(1, _D))
        mp = _segsum(h, src, dst, zeros)
        h = _gru(
            mp, h,
            Wih[i0], bih[i0].reshape(1, 3 * _D),
            Whh[i0], bhh[i0].reshape(1, 3 * _D),
            Wlin[i1], blin[i1].reshape(1, _D),
        )
        mp = _segsum(h, src, dst, zeros)
        outs.append(
            _gru(
                mp, h,
                Wih[i1], bih[i1].reshape(1, 3 * _D),
                Whh[i1], bhh[i1].reshape(1, 3 * _D),
                fcW, fcb.reshape(1, _D),
            )
        )
    return jnp.concatenate(outs, axis=0)


# speedup vs baseline: 1.4566x; 1.0088x over previous
import functools

import jax
import jax.numpy as jnp
from jax import lax
from jax.experimental import pallas as pl
from jax.experimental.pallas import tpu as pltpu
from jax.experimental.pallas import tpu_sc as plsc

_N = 10000
_D = 128
_E = 320000
_NC = 2
_NS = 16
_NW = _NC * _NS
_EPW = _E // _NW
_K = 80
_NCH = _EPW // _K
_CPPS = (32, 32, 32, 29)
_CPP0 = _CPPS[0]
_RPT = 624
_RTAIL = _N - _NS * _RPT
_BLK = 1000
_GRID = _N // _BLK

def _segsum_body(h_hbm, src_hbm, dst_hbm, zeros_hbm, out_hbm, src_v, dst_v, rows_v, m_sh, gsem, ssem):
    c = lax.axis_index("c")
    s = lax.axis_index("s")
    wid = c * _NS + s
    pltpu.sync_copy(zeros_hbm.at[pl.ds(0, _RPT)], m_sh.at[pl.ds(s * _RPT, _RPT)])

    @pl.when(s == _NS - 1)
    def _():
        pltpu.sync_copy(
            zeros_hbm.at[pl.ds(_RPT, _RTAIL)],
            m_sh.at[pl.ds(_NS * _RPT, _RTAIL)],
        )
    pltpu.sync_copy(src_hbm.at[wid, pl.ds(0, _CPP0)], src_v.at[pl.ds(0, _CPP0)])
    pltpu.sync_copy(dst_hbm.at[wid, pl.ds(0, _CPP0)], dst_v.at[pl.ds(0, _CPP0)])
    plsc.subcore_barrier()

    def _buf(j):
        return jnp.bitwise_and(j, 3)

    def _issue_gather(j):
        b = _buf(j)
        pltpu.async_copy(h_hbm.at[src_v.at[j]], rows_v.at[b], gsem.at[b])

    def _wait_gather(j):
        b = _buf(j)
        pltpu.make_async_copy(h_hbm.at[src_v.at[j]], rows_v.at[b], gsem.at[b]).wait()

    def _issue_scatter(j):
        b = _buf(j)
        pltpu.async_copy(rows_v.at[b], m_sh.at[dst_v.at[j]], ssem.at[b], add=True)

    def _wait_scatter(j):
        b = _buf(j)
        pltpu.make_async_copy(rows_v.at[b], m_sh.at[dst_v.at[j]], ssem.at[b]).wait()

    def body(j, carry):
        _wait_gather(j)

        @pl.when(j >= 1)
        def _():
            _wait_scatter(j - 1)

        @pl.when(j + 3 < carry)
        def _():
            _issue_gather(j + 3)

        _issue_scatter(j)
        return carry

    base = 0
    for p, cpp in enumerate(_CPPS):
        if p > 0:
            pltpu.sync_copy(
                src_hbm.at[wid, pl.ds(base, cpp)], src_v.at[pl.ds(0, cpp)]
            )
            pltpu.sync_copy(
                dst_hbm.at[wid, pl.ds(base, cpp)], dst_v.at[pl.ds(0, cpp)]
            )
        base += cpp
        _issue_gather(jnp.int32(0))
        _issue_gather(jnp.int32(1))
        _issue_gather(jnp.int32(2))
        lax.fori_loop(0, cpp, body, jnp.int32(cpp))
        _wait_scatter(jnp.int32(cpp - 1))
    plsc.subcore_barrier()
    pltpu.sync_copy(m_sh.at[pl.ds(s * _RPT, _RPT)], out_hbm.at[c, pl.ds(s * _RPT, _RPT)])

    @pl.when(s == _NS - 1)
    def _():
        pltpu.sync_copy(
            m_sh.at[pl.ds(_NS * _RPT, _RTAIL)],
            out_hbm.at[c, pl.ds(_NS * _RPT, _RTAIL)],
        )


@functools.cache
def _make_segsum():
    mesh = plsc.VectorSubcoreMesh(
        core_axis_name="c", subcore_axis_name="s", num_cores=_NC, num_subcores=_NS
    )
    return pl.kernel(
        _segsum_body,
        out_type=jax.ShapeDtypeStruct((_NC, _N, _D), jnp.float32),
        mesh=mesh,
        scratch_types=[
            pltpu.VMEM((_CPP0, _K), jnp.int32),
            pltpu.VMEM((_CPP0, _K), jnp.int32),
            pltpu.VMEM((4, _K, _D), jnp.float32),
            pltpu.VMEM_SHARED((_N, _D), jnp.float32),
            pltpu.SemaphoreType.DMA((4,)),
            pltpu.SemaphoreType.DMA((4,)),
        ],
    )


def _proj_body(x_ref, w_ref, b_ref, o_ref):
    o_ref[...] = (
        jnp.dot(x_ref[...], w_ref[...], preferred_element_type=jnp.float32) + b_ref[...]
    )


_proj = pl.pallas_call(
    _proj_body,
    grid=(_GRID,),
    in_specs=[
        pl.BlockSpec((_BLK, _D), lambda i: (i, 0)),
        pl.BlockSpec((_D, _D), lambda i: (0, 0)),
        pl.BlockSpec((1, _D), lambda i: (0, 0)),
    ],
    out_specs=pl.BlockSpec((_BLK, _D), lambda i: (i, 0)),
    out_shape=jax.ShapeDtypeStruct((_N, _D), jnp.float32),
)


def _gru_body(mp_ref, h_ref, wih_ref, bih_ref, whh_ref, bhh_ref, wn_ref, bn_ref, o_ref):
    m = mp_ref[0] + mp_ref[1]
    h = h_ref[...]
    gi = jnp.dot(m, wih_ref[...], preferred_element_type=jnp.float32) + bih_ref[...]
    gh = jnp.dot(h, whh_ref[...], preferred_element_type=jnp.float32) + bhh_ref[...]
    r = jax.nn.sigmoid(gi[:, :_D] + gh[:, :_D])
    z = jax.nn.sigmoid(gi[:, _D:2 * _D] + gh[:, _D:2 * _D])
    n = jnp.tanh(gi[:, 2 * _D:] + r * gh[:, 2 * _D:])
    x = (1.0 - z) * n + z * h
    e = jnp.where(x > 0, x, jnp.exp(x) - 1.0)
    o_ref[...] = (
        jnp.dot(e, wn_ref[...], preferred_element_type=jnp.float32) + bn_ref[...]
    )


_gru = pl.pallas_call(
    _gru_body,
    grid=(_GRID,),
    in_specs=[
        pl.BlockSpec((_NC, _BLK, _D), lambda i: (0, i, 0)),
        pl.BlockSpec((_BLK, _D), lambda i: (i, 0)),
        pl.BlockSpec((_D, 3 * _D), lambda i: (0, 0)),
        pl.BlockSpec((1, 3 * _D), lambda i: (0, 0)),
        pl.BlockSpec((_D, 3 * _D), lambda i: (0, 0)),
        pl.BlockSpec((1, 3 * _D), lambda i: (0, 0)),
        pl.BlockSpec((_D, _D), lambda i: (0, 0)),
        pl.BlockSpec((1, _D), lambda i: (0, 0)),
    ],
    out_specs=pl.BlockSpec((_BLK, _D), lambda i: (i, 0)),
    out_shape=jax.ShapeDtypeStruct((_N, _D), jnp.float32),
)


def kernel(x_0, edge_index_0, x_1, edge_index_1, Wlin, blin, Wih, bih, Whh, bhh, fcW, fcb):
    zeros = jnp.zeros((_RPT + _RTAIL, _D), jnp.float32)
    _segsum = _make_segsum()
    outs = []
    for g, (x, ei) in enumerate(((x_0, edge_index_0), (x_1, edge_index_1))):
        src = ei[0].reshape(_NW, _NCH, _K)
        dst = ei[1].reshape(_NW, _NCH, _K)
        i0, i1 = 2 * g, 2 * g + 1
        h = _proj(x, Wlin[i0], blin[i0].reshape(1, _D))
        mp = _segsum(h, src, dst, zeros)
        h = _gru(
            mp, h,
            Wih[i0], bih[i0].reshape(1, 3 * _D),
            Whh[i0], bhh[i0].reshape(1, 3 * _D),
            Wlin[i1], blin[i1].reshape(1, _D),
        )
        mp = _segsum(h, src, dst, zeros)
        outs.append(
            _gru(
                mp, h,
                Wih[i1], bih[i1].reshape(1, 3 * _D),
                Whh[i1], bhh[i1].reshape(1, 3 * _D),
                fcW, fcb.reshape(1, _D),
            )
        )
    return jnp.concatenate(outs, axis=0)


# speedup vs baseline: 1.4870x; 1.0209x over previous
import functools

import jax
import jax.numpy as jnp
from jax import lax
from jax.experimental import pallas as pl
from jax.experimental.pallas import tpu as pltpu
from jax.experimental.pallas import tpu_sc as plsc

_N = 10000
_D = 128
_E = 320000
_NC = 2
_NS = 16
_NW = _NC * _NS
_EPW = _E // _NW
_K = 80
_NCH = _EPW // _K
_CPPS = (32, 32, 32, 29)
_CPP0 = _CPPS[0]
_RPT = 624
_RTAIL = _N - _NS * _RPT
_BLK = 1000
_GRID = _N // _BLK

def _segsum_body(h_hbm, src_hbm, dst_hbm, zeros_hbm, out_hbm, src_v, dst_v, rows_v, m_sh, gsem, ssem, zsem):
    c = lax.axis_index("c")
    s = lax.axis_index("s")
    wid = c * _NS + s
    zcopy = pltpu.async_copy(
        zeros_hbm.at[pl.ds(0, _RPT)], m_sh.at[pl.ds(s * _RPT, _RPT)], zsem
    )
    pltpu.sync_copy(src_hbm.at[wid, pl.ds(0, _CPP0)], src_v.at[pl.ds(0, _CPP0)])
    pltpu.sync_copy(dst_hbm.at[wid, pl.ds(0, _CPP0)], dst_v.at[pl.ds(0, _CPP0)])

    @pl.when(s == _NS - 1)
    def _():
        pltpu.sync_copy(
            zeros_hbm.at[pl.ds(_RPT, _RTAIL)],
            m_sh.at[pl.ds(_NS * _RPT, _RTAIL)],
        )

    def _buf(j):
        return jnp.bitwise_and(j, 3)

    def _issue_gather(j):
        b = _buf(j)
        pltpu.async_copy(h_hbm.at[src_v.at[j]], rows_v.at[b], gsem.at[b])

    def _wait_gather(j):
        b = _buf(j)
        pltpu.make_async_copy(h_hbm.at[src_v.at[j]], rows_v.at[b], gsem.at[b]).wait()

    def _issue_scatter(j):
        b = _buf(j)
        pltpu.async_copy(rows_v.at[b], m_sh.at[dst_v.at[j]], ssem.at[b], add=True)

    def _wait_scatter(j):
        b = _buf(j)
        pltpu.make_async_copy(rows_v.at[b], m_sh.at[dst_v.at[j]], ssem.at[b]).wait()

    def body(j, carry):
        _wait_gather(j)
        _wait_scatter(j - 1)
        _issue_gather(j + 3)
        _issue_scatter(j)
        return carry

    base = 0
    first = True
    for p, cpp in enumerate(_CPPS):
        if p > 0:
            pltpu.sync_copy(
                src_hbm.at[wid, pl.ds(base, cpp)], src_v.at[pl.ds(0, cpp)]
            )
            pltpu.sync_copy(
                dst_hbm.at[wid, pl.ds(base, cpp)], dst_v.at[pl.ds(0, cpp)]
            )
        base += cpp
        _issue_gather(jnp.int32(0))
        _issue_gather(jnp.int32(1))
        _issue_gather(jnp.int32(2))
        if first:
            zcopy.wait()
            plsc.subcore_barrier()
            first = False
        _wait_gather(jnp.int32(0))
        _issue_gather(jnp.int32(3))
        _issue_scatter(jnp.int32(0))
        lax.fori_loop(1, cpp - 3, body, 0)
        for j in (cpp - 3, cpp - 2, cpp - 1):
            _wait_gather(jnp.int32(j))
            _wait_scatter(jnp.int32(j - 1))
            _issue_scatter(jnp.int32(j))
        _wait_scatter(jnp.int32(cpp - 1))
    plsc.subcore_barrier()
    pltpu.sync_copy(m_sh.at[pl.ds(s * _RPT, _RPT)], out_hbm.at[c, pl.ds(s * _RPT, _RPT)])

    @pl.when(s == _NS - 1)
    def _():
        pltpu.sync_copy(
            m_sh.at[pl.ds(_NS * _RPT, _RTAIL)],
            out_hbm.at[c, pl.ds(_NS * _RPT, _RTAIL)],
        )


@functools.cache
def _make_segsum():
    mesh = plsc.VectorSubcoreMesh(
        core_axis_name="c", subcore_axis_name="s", num_cores=_NC, num_subcores=_NS
    )
    return pl.kernel(
        _segsum_body,
        out_type=jax.ShapeDtypeStruct((_NC, _N, _D), jnp.float32),
        mesh=mesh,
        scratch_types=[
            pltpu.VMEM((_CPP0, _K), jnp.int32),
            pltpu.VMEM((_CPP0, _K), jnp.int32),
            pltpu.VMEM((4, _K, _D), jnp.float32),
            pltpu.VMEM_SHARED((_N, _D), jnp.float32),
            pltpu.SemaphoreType.DMA((4,)),
            pltpu.SemaphoreType.DMA((4,)),
            pltpu.SemaphoreType.DMA,
        ],
    )


def _proj_body(x_ref, w_ref, b_ref, o_ref):
    o_ref[...] = (
        jnp.dot(x_ref[...], w_ref[...], preferred_element_type=jnp.float32) + b_ref[...]
    )


_proj = pl.pallas_call(
    _proj_body,
    grid=(_GRID,),
    in_specs=[
        pl.BlockSpec((_BLK, _D), lambda i: (i, 0)),
        pl.BlockSpec((_D, _D), lambda i: (0, 0)),
        pl.BlockSpec((1, _D), lambda i: (0, 0)),
    ],
    out_specs=pl.BlockSpec((_BLK, _D), lambda i: (i, 0)),
    out_shape=jax.ShapeDtypeStruct((_N, _D), jnp.float32),
)


def _gru_body(mp_ref, h_ref, wih_ref, bih_ref, whh_ref, bhh_ref, wn_ref, bn_ref, o_ref):
    m = mp_ref[0] + mp_ref[1]
    h = h_ref[...]
    gi = jnp.dot(m, wih_ref[...], preferred_element_type=jnp.float32) + bih_ref[...]
    gh = jnp.dot(h, whh_ref[...], preferred_element_type=jnp.float32) + bhh_ref[...]
    r = jax.nn.sigmoid(gi[:, :_D] + gh[:, :_D])
    z = jax.nn.sigmoid(gi[:, _D:2 * _D] + gh[:, _D:2 * _D])
    n = jnp.tanh(gi[:, 2 * _D:] + r * gh[:, 2 * _D:])
    x = (1.0 - z) * n + z * h
    e = jnp.where(x > 0, x, jnp.exp(x) - 1.0)
    o_ref[...] = (
        jnp.dot(e, wn_ref[...], preferred_element_type=jnp.float32) + bn_ref[...]
    )


_gru = pl.pallas_call(
    _gru_body,
    grid=(_GRID,),
    in_specs=[
        pl.BlockSpec((_NC, _BLK, _D), lambda i: (0, i, 0)),
        pl.BlockSpec((_BLK, _D), lambda i: (i, 0)),
        pl.BlockSpec((_D, 3 * _D), lambda i: (0, 0)),
        pl.BlockSpec((1, 3 * _D), lambda i: (0, 0)),
        pl.BlockSpec((_D, 3 * _D), lambda i: (0, 0)),
        pl.BlockSpec((1, 3 * _D), lambda i: (0, 0)),
        pl.BlockSpec((_D, _D), lambda i: (0, 0)),
        pl.BlockSpec((1, _D), lambda i: (0, 0)),
    ],
    out_specs=pl.BlockSpec((_BLK, _D), lambda i: (i, 0)),
    out_shape=jax.ShapeDtypeStruct((_N, _D), jnp.float32),
)


def kernel(x_0, edge_index_0, x_1, edge_index_1, Wlin, blin, Wih, bih, Whh, bhh, fcW, fcb):
    zeros = jnp.zeros((_RPT + _RTAIL, _D), jnp.float32)
    _segsum = _make_segsum()
    outs = []
    for g, (x, ei) in enumerate(((x_0, edge_index_0), (x_1, edge_index_1))):
        src = ei[0].reshape(_NW, _NCH, _K)
        dst = ei[1].reshape(_NW, _NCH, _K)
        i0, i1 = 2 * g, 2 * g + 1
        h = _proj(x, Wlin[i0], blin[i0].reshape(1, _D))
        mp = _segsum(h, src, dst, zeros)
        h = _gru(
            mp, h,
            Wih[i0], bih[i0].reshape(1, 3 * _D),
            Whh[i0], bhh[i0].reshape(1, 3 * _D),
            Wlin[i1], blin[i1].reshape(1, _D),
        )
        mp = _segsum(h, src, dst, zeros)
        outs.append(
            _gru(
                mp, h,
                Wih[i1], bih[i1].reshape(1, 3 * _D),
                Whh[i1], bhh[i1].reshape(1, 3 * _D),
                fcW, fcb.reshape(1, _D),
            )
        )
    return jnp.concatenate(outs, axis=0)


# speedup vs baseline: 1.4890x; 1.0013x over previous
import functools

import jax
import jax.numpy as jnp
from jax import lax
from jax.experimental import pallas as pl
from jax.experimental.pallas import tpu as pltpu
from jax.experimental.pallas import tpu_sc as plsc

_N = 10000
_D = 128
_E = 320000
_NC = 2
_NS = 16
_NW = _NC * _NS
_EPW = _E // _NW
_K = 80
_NCH = _EPW // _K
_CPPS = (32, 32, 32, 29)
_CPP0 = _CPPS[0]
_RPT = 624
_RTAIL = _N - _NS * _RPT
_BLK = 1000
_GRID = _N // _BLK

def _segsum_body(h_hbm, src_hbm, dst_hbm, zeros_hbm, out_hbm, src_v, dst_v, rows_v, m_sh, gsem, ssem, zsem):
    c = lax.axis_index("c")
    s = lax.axis_index("s")
    wid = c * _NS + s
    zcopy = pltpu.async_copy(
        zeros_hbm.at[pl.ds(0, _RPT)], m_sh.at[pl.ds(s * _RPT, _RPT)], zsem
    )
    pltpu.sync_copy(src_hbm.at[wid, pl.ds(0, _CPP0)], src_v.at[pl.ds(0, _CPP0)])
    pltpu.sync_copy(dst_hbm.at[wid, pl.ds(0, _CPP0)], dst_v.at[pl.ds(0, _CPP0)])

    @pl.when(s == _NS - 1)
    def _():
        pltpu.sync_copy(
            zeros_hbm.at[pl.ds(_RPT, _RTAIL)],
            m_sh.at[pl.ds(_NS * _RPT, _RTAIL)],
        )

    def _buf(j):
        return jnp.bitwise_and(j, 3)

    def _issue_gather(j):
        b = _buf(j)
        pltpu.async_copy(h_hbm.at[src_v.at[j]], rows_v.at[b], gsem.at[b])

    def _wait_gather(j):
        b = _buf(j)
        pltpu.make_async_copy(h_hbm.at[src_v.at[j]], rows_v.at[b], gsem.at[b]).wait()

    def _issue_scatter(j):
        b = _buf(j)
        pltpu.async_copy(rows_v.at[b], m_sh.at[dst_v.at[j]], ssem.at[b], add=True)

    def _wait_scatter(j):
        b = _buf(j)
        pltpu.make_async_copy(rows_v.at[b], m_sh.at[dst_v.at[j]], ssem.at[b]).wait()

    def body(j, carry):
        _wait_gather(j)
        _wait_scatter(j - 1)
        _issue_gather(j + 3)
        _issue_scatter(j)
        return carry

    base = 0
    first = True
    for p, cpp in enumerate(_CPPS):
        if p > 0:
            pltpu.sync_copy(
                src_hbm.at[wid, pl.ds(base, cpp)], src_v.at[pl.ds(0, cpp)]
            )
            pltpu.sync_copy(
                dst_hbm.at[wid, pl.ds(base, cpp)], dst_v.at[pl.ds(0, cpp)]
            )
        base += cpp
        _issue_gather(jnp.int32(0))
        _issue_gather(jnp.int32(1))
        _issue_gather(jnp.int32(2))
        if first:
            zcopy.wait()
            plsc.subcore_barrier()
            first = False
        _wait_gather(jnp.int32(0))
        _issue_gather(jnp.int32(3))
        _issue_scatter(jnp.int32(0))
        lax.fori_loop(1, cpp - 3, body, 0)
        for j in (cpp - 3, cpp - 2, cpp - 1):
            _wait_gather(jnp.int32(j))
            _wait_scatter(jnp.int32(j - 1))
            _issue_scatter(jnp.int32(j))
        _wait_scatter(jnp.int32(cpp - 1))
    plsc.subcore_barrier()
    pltpu.sync_copy(m_sh.at[pl.ds(s * _RPT, _RPT)], out_hbm.at[c, pl.ds(s * _RPT, _RPT)])

    @pl.when(s == _NS - 1)
    def _():
        pltpu.sync_copy(
            m_sh.at[pl.ds(_NS * _RPT, _RTAIL)],
            out_hbm.at[c, pl.ds(_NS * _RPT, _RTAIL)],
        )


@functools.cache
def _make_segsum():
    mesh = plsc.VectorSubcoreMesh(
        core_axis_name="c", subcore_axis_name="s", num_cores=_NC, num_subcores=_NS
    )
    return pl.kernel(
        _segsum_body,
        out_type=jax.ShapeDtypeStruct((_NC, _N, _D), jnp.float32),
        mesh=mesh,
        scratch_types=[
            pltpu.VMEM((_CPP0, _K), jnp.int32),
            pltpu.VMEM((_CPP0, _K), jnp.int32),
            pltpu.VMEM((4, _K, _D), jnp.float32),
            pltpu.VMEM_SHARED((_N, _D), jnp.float32),
            pltpu.SemaphoreType.DMA((4,)),
            pltpu.SemaphoreType.DMA((4,)),
            pltpu.SemaphoreType.DMA,
        ],
    )


def _proj_body(x_ref, w_ref, b_ref, o_ref):
    o_ref[...] = (
        jnp.dot(x_ref[...], w_ref[...], preferred_element_type=jnp.float32) + b_ref[...]
    )


_proj = pl.pallas_call(
    _proj_body,
    grid=(_GRID,),
    in_specs=[
        pl.BlockSpec((_BLK, _D), lambda i: (i, 0)),
        pl.BlockSpec((_D, _D), lambda i: (0, 0)),
        pl.BlockSpec((1, _D), lambda i: (0, 0)),
    ],
    out_specs=pl.BlockSpec((_BLK, _D), lambda i: (i, 0)),
    out_shape=jax.ShapeDtypeStruct((_N, _D), jnp.float32),
)


def _gru_body(mp_ref, h_ref, wih_ref, bih_ref, whh_ref, bhh_ref, wn_ref, bn_ref, o_ref):
    m = mp_ref[0] + mp_ref[1]
    h = h_ref[...]
    gi = jnp.dot(m, wih_ref[...], preferred_element_type=jnp.float32) + bih_ref[...]
    gh = jnp.dot(h, whh_ref[...], preferred_element_type=jnp.float32) + bhh_ref[...]
    r = jax.nn.sigmoid(gi[:, :_D] + gh[:, :_D])
    z = jax.nn.sigmoid(gi[:, _D:2 * _D] + gh[:, _D:2 * _D])
    n = jnp.tanh(gi[:, 2 * _D:] + r * gh[:, 2 * _D:])
    x = (1.0 - z) * n + z * h
    e = jnp.where(x > 0, x, jnp.exp(x) - 1.0)
    o_ref[...] = (
        jnp.dot(e, wn_ref[...], preferred_element_type=jnp.float32) + bn_ref[...]
    )


_gru = pl.pallas_call(
    _gru_body,
    grid=(_GRID,),
    in_specs=[
        pl.BlockSpec((_NC, _BLK, _D), lambda i: (0, i, 0)),
        pl.BlockSpec((_BLK, _D), lambda i: (i, 0)),
        pl.BlockSpec((_D, 3 * _D), lambda i: (0, 0)),
        pl.BlockSpec((1, 3 * _D), lambda i: (0, 0)),
        pl.BlockSpec((_D, 3 * _D), lambda i: (0, 0)),
        pl.BlockSpec((1, 3 * _D), lambda i: (0, 0)),
        pl.BlockSpec((_D, _D), lambda i: (0, 0)),
        pl.BlockSpec((1, _D), lambda i: (0, 0)),
    ],
    out_specs=pl.BlockSpec((_BLK, _D), lambda i: (i, 0)),
    out_shape=jax.ShapeDtypeStruct((_N, _D), jnp.float32),
)


def kernel(x_0, edge_index_0, x_1, edge_index_1, Wlin, blin, Wih, bih, Whh, bhh, fcW, fcb):
    zeros = jnp.zeros((_RPT + _RTAIL, _D), jnp.float32)
    _segsum = _make_segsum()
    src = [None, None]
    dst = [None, None]
    h = [None, None]
    mp = [None, None]
    for g, ei in enumerate((edge_index_0, edge_index_1)):
        src[g] = ei[0].reshape(_NW, _NCH, _K)
        dst[g] = ei[1].reshape(_NW, _NCH, _K)
    for g, x in enumerate((x_0, x_1)):
        i0 = 2 * g
        h[g] = _proj(x, Wlin[i0], blin[i0].reshape(1, _D))
    for g in range(2):
        mp[g] = _segsum(h[g], src[g], dst[g], zeros)
    for g in range(2):
        i0, i1 = 2 * g, 2 * g + 1
        h[g] = _gru(
            mp[g], h[g],
            Wih[i0], bih[i0].reshape(1, 3 * _D),
            Whh[i0], bhh[i0].reshape(1, 3 * _D),
            Wlin[i1], blin[i1].reshape(1, _D),
        )
        mp[g] = _segsum(h[g], src[g], dst[g], zeros)
    outs = []
    for g in range(2):
        i1 = 2 * g + 1
        outs.append(
            _gru(
                mp[g], h[g],
                Wih[i1], bih[i1].reshape(1, 3 * _D),
                Whh[i1], bhh[i1].reshape(1, 3 * _D),
                fcW, fcb.reshape(1, _D),
            )
        )
    return jnp.concatenate(outs, axis=0)
